# baseline (speedup 1.0000x reference)
import jax
import jax.numpy as jnp
from jax import lax
from jax.experimental import pallas as pl
from jax.experimental.pallas import tpu as pltpu
from jax.experimental.pallas import tpu_sc as plsc


def _mlp_table_body(al_ref, w1_ref, b1_ref, w2_ref, b2_ref, con_ref, out_ref):
    alpha = jax.nn.sigmoid(al_ref[0, 0])
    c = con_ref[:, :]
    h = lax.dot_general(c, w1_ref[:, :], (((1,), (1,)), ((), ())),
                        preferred_element_type=jnp.float32)
    h = jnp.maximum(h + b1_ref[:, :], 0.0)
    m = lax.dot_general(h, w2_ref[:, :], (((1,), (1,)), ((), ())),
                        preferred_element_type=jnp.float32)
    m = alpha * (m + b2_ref[:, :])
    out_ref[:, :] = jnp.concatenate([m, m], axis=1)


def _make_mlp_table(concept_table, alpha_logit, W1, b1, W2, b2):
    V, C = concept_table.shape
    RB = 10000
    assert V % RB == 0
    grid = (V // RB,)
    full = lambda shape: pl.BlockSpec(shape, lambda i: (0,) * len(shape))
    return pl.pallas_call(
        _mlp_table_body,
        grid=grid,
        in_specs=[full((1, 1)), full((C, C)), full((1, C)),
                  full((C, C)), full((1, C)),
                  pl.BlockSpec((RB, C), lambda i: (i, 0))],
        out_specs=pl.BlockSpec((RB, 2 * C), lambda i: (i, 0)),
        out_shape=jax.ShapeDtypeStruct((V, 2 * C), jnp.float32),
    )(alpha_logit.reshape(1, 1), W1, b1.reshape(1, C), W2, b2.reshape(1, C),
      concept_table)


def _gather_combine(sym, mlp2, law, idx2, pos, beta_v, B, T, C, NW, mesh):
    FPW = B // NW
    CH = T // 100
    assert FPW % 2 == 0

    def body(sym_hbm, mlp_hbm, law_hbm, idx_hbm, pos_hbm, beta_hbm, out_hbm,
             idx_a, idx_b, s_a, s_b, m_a, m_b, l_a, l_b, pos_v, beta_sc,
             sg_a, sg_b, so_a, so_b, si_a, si_b):
        wid = lax.axis_index("c") * mesh.num_subcores + lax.axis_index("s")
        base = wid * FPW
        pltpu.sync_copy(pos_hbm, pos_v)
        pltpu.sync_copy(beta_hbm, beta_sc)
        beta = 1.0 / (1.0 + jnp.exp(-beta_sc[:]))

        idx_ref = [idx_a, idx_b]
        s_ref = [s_a, s_b]
        m_ref = [m_a, m_b]
        l_ref = [l_a, l_b]
        sg = [sg_a, sg_b]
        so = [so_a, so_b]
        si = [si_a, si_b]

        def start_gather(b):
            for k in range(CH):
                sl = pl.ds(k * 100, 100)
                pltpu.async_copy(sym_hbm.at[idx_ref[b].at[k]],
                                 s_ref[b].at[sl], sg[b])
                pltpu.async_copy(mlp_hbm.at[idx_ref[b].at[k]],
                                 m_ref[b].at[sl], sg[b])
                pltpu.async_copy(law_hbm.at[idx_ref[b].at[k]],
                                 l_ref[b].at[sl], sg[b])

        def wait_gather(b):
            for k in range(CH):
                sl = pl.ds(k * 100, 100)
                pltpu.make_async_copy(sym_hbm.at[idx_ref[b].at[k]],
                                      s_ref[b].at[sl], sg[b]).wait()
                pltpu.make_async_copy(mlp_hbm.at[idx_ref[b].at[k]],
                                      m_ref[b].at[sl], sg[b]).wait()
                pltpu.make_async_copy(law_hbm.at[idx_ref[b].at[k]],
                                      l_ref[b].at[sl], sg[b]).wait()

        def start_idx(f, b):
            pltpu.async_copy(idx_hbm.at[pl.ds(f * CH, CH)], idx_ref[b], si[b])

        def wait_idx(b):
            pltpu.make_async_copy(idx_hbm.at[pl.ds(0, CH)],
                                  idx_ref[b], si[b]).wait()

        def combine(b):
            def row_body(r, carry2):
                for c2 in range(C // 16):
                    sl = pl.ds(c2 * 16, 16)
                    t = m_ref[b][r, sl] + (l_ref[b][r, sl] * beta
                                           + pos_v[r, sl])
                    plsc.addupdate(s_ref[b].at[r, sl], t)
                return carry2
            lax.fori_loop(0, T, row_body, 0, unroll=2)

        def start_out(f, b):
            pltpu.async_copy(s_ref[b], out_hbm.at[pl.ds(f * T, T)], so[b])

        def wait_out(b):
            pltpu.make_async_copy(s_ref[b],
                                  out_hbm.at[pl.ds(0, T)], so[b]).wait()

        pltpu.sync_copy(idx_hbm.at[pl.ds(base * CH, CH)], idx_a)
        start_gather(0)
        start_idx(base + 1, 1)

        def pair_body(jj, carry):
            j = jj * 2

            def half(j, b):
                f = base + j
                nb = 1 - b
                @pl.when(j + 1 < FPW)
                def _():
                    wait_idx(nb)
                    @pl.when(j >= 1)
                    def _():
                        wait_out(nb)
                    start_gather(nb)
                wait_gather(b)
                @pl.when(j + 2 < FPW)
                def _():
                    start_idx(f + 2, b)
                combine(b)
                start_out(f, b)

            half(j, 0)
            half(j + 1, 1)
            return carry

        lax.fori_loop(0, FPW // 2, pair_body, 0)
        wait_out(0)
        wait_out(1)

    k = pl.kernel(
        body,
        out_type=jax.ShapeDtypeStruct((B * T, C), jnp.float32),
        mesh=mesh,
        compiler_params=pltpu.CompilerParams(use_tc_tiling_on_sc=False),
        scratch_types=[
            pltpu.VMEM((CH, 100), jnp.int32),
            pltpu.VMEM((CH, 100), jnp.int32),
            pltpu.VMEM((T, C), jnp.float32),
            pltpu.VMEM((T, C), jnp.float32),
            pltpu.VMEM((T, 2 * C), jnp.float32),
            pltpu.VMEM((T, 2 * C), jnp.float32),
            pltpu.VMEM((T, C), jnp.float32),
            pltpu.VMEM((T, C), jnp.float32),
            pltpu.VMEM((T, C), jnp.float32),
            pltpu.VMEM((16,), jnp.float32),
            pltpu.SemaphoreType.DMA,
            pltpu.SemaphoreType.DMA,
            pltpu.SemaphoreType.DMA,
            pltpu.SemaphoreType.DMA,
            pltpu.SemaphoreType.DMA,
            pltpu.SemaphoreType.DMA,
        ],
    )
    return k(sym, mlp2, law, idx2, pos, beta_v)


def kernel(idx, symbol_table, concept_table, law_table, pos_table,
           alpha_logit, beta_logit, W1, b1, W2, b2):
    B, T = idx.shape
    V, C = symbol_table.shape
    mlp2 = _make_mlp_table(concept_table, alpha_logit, W1, b1, W2, b2)
    mesh = plsc.VectorSubcoreMesh(core_axis_name="c", subcore_axis_name="s")
    NW = mesh.num_cores * mesh.num_subcores
    assert T % 100 == 0 and B % NW == 0 and C % 16 == 0
    idx2 = idx.reshape(B * T // 100, 100)
    pos = pos_table[:T]
    beta_v = jnp.broadcast_to(beta_logit.reshape(1), (16,))
    out = _gather_combine(symbol_table, mlp2, law_table, idx2, pos, beta_v,
                          B, T, C, NW, mesh)
    return out.reshape(B, T, C)


# speedup vs baseline: 1.1280x; 1.1280x over previous
import jax
import jax.numpy as jnp
from jax import lax
from jax.experimental import pallas as pl
from jax.experimental.pallas import tpu as pltpu
from jax.experimental.pallas import tpu_sc as plsc


def _fused_table_body(al_ref, be_ref, w1_ref, b1_ref, w2_ref, b2_ref,
                      sym_ref, con_ref, law_ref, out_ref):
    alpha = jax.nn.sigmoid(al_ref[0, 0])
    beta = jax.nn.sigmoid(be_ref[0, 0])
    c = con_ref[:, :]
    h = lax.dot_general(c, w1_ref[:, :], (((1,), (1,)), ((), ())),
                        preferred_element_type=jnp.float32)
    h = jnp.maximum(h + b1_ref[:, :], 0.0)
    cr = lax.dot_general(h, w2_ref[:, :], (((1,), (1,)), ((), ())),
                         preferred_element_type=jnp.float32)
    cr = cr + b2_ref[:, :]
    f = sym_ref[:, :] + alpha * cr + beta * law_ref[:, :]
    out_ref[:, :] = jnp.concatenate([f, f], axis=1)


def _make_fused_table(symbol_table, concept_table, law_table,
                      alpha_logit, beta_logit, W1, b1, W2, b2):
    V, C = symbol_table.shape
    RB = 10000
    assert V % RB == 0
    grid = (V // RB,)
    tab = pl.BlockSpec((RB, C), lambda i: (i, 0))
    full = lambda shape: pl.BlockSpec(shape, lambda i: (0,) * len(shape))
    return pl.pallas_call(
        _fused_table_body,
        grid=grid,
        in_specs=[full((1, 1)), full((1, 1)),
                  full((C, C)), full((1, C)), full((C, C)), full((1, C)),
                  tab, tab, tab],
        out_specs=pl.BlockSpec((RB, 2 * C), lambda i: (i, 0)),
        out_shape=jax.ShapeDtypeStruct((V, 2 * C), jnp.float32),
    )(alpha_logit.reshape(1, 1), beta_logit.reshape(1, 1),
      W1, b1.reshape(1, C), W2, b2.reshape(1, C),
      symbol_table, concept_table, law_table)


def _gather_pos(fused2, idx2, pos, B, T, C, NW, mesh):
    FPW = B // NW
    CH = T // 100
    assert FPW % 2 == 0

    def body(fused_hbm, idx_hbm, pos_hbm, out_hbm,
             idx_a, idx_b, rows_a, rows_b, o_a, o_b, pos_v,
             sg_a, sg_b, so_a, so_b, si_a, si_b):
        wid = lax.axis_index("c") * mesh.num_subcores + lax.axis_index("s")
        base = wid * FPW
        pltpu.sync_copy(pos_hbm, pos_v)

        idx_ref = [idx_a, idx_b]
        rows_ref = [rows_a, rows_b]
        o_ref = [o_a, o_b]
        sg = [sg_a, sg_b]
        so = [so_a, so_b]
        si = [si_a, si_b]

        def start_gather(b):
            for k in range(CH):
                pltpu.async_copy(fused_hbm.at[idx_ref[b].at[k]],
                                 rows_ref[b].at[pl.ds(k * 100, 100)], sg[b])

        def wait_gather(b):
            for k in range(CH):
                pltpu.make_async_copy(fused_hbm.at[idx_ref[b].at[k]],
                                      rows_ref[b].at[pl.ds(k * 100, 100)],
                                      sg[b]).wait()

        def start_idx(f, b):
            pltpu.async_copy(idx_hbm.at[pl.ds(f * CH, CH)], idx_ref[b], si[b])

        def wait_idx(b):
            pltpu.make_async_copy(idx_hbm.at[pl.ds(0, CH)],
                                  idx_ref[b], si[b]).wait()

        def combine(b):
            def row_body(r, carry2):
                for c2 in range(C // 16):
                    sl = pl.ds(c2 * 16, 16)
                    o_ref[b][r, sl] = rows_ref[b][r, sl] + pos_v[r, sl]
                return carry2
            lax.fori_loop(0, T, row_body, 0, unroll=2)

        def start_out(f, b):
            pltpu.async_copy(o_ref[b], out_hbm.at[pl.ds(f * T, T)], so[b])

        def wait_out(b):
            pltpu.make_async_copy(o_ref[b],
                                  out_hbm.at[pl.ds(0, T)], so[b]).wait()

        pltpu.sync_copy(idx_hbm.at[pl.ds(base * CH, CH)], idx_a)
        start_gather(0)
        start_idx(base + 1, 1)

        def pair_body(jj, carry):
            j = jj * 2

            def half(j, b):
                f = base + j
                nb = 1 - b
                @pl.when(j + 1 < FPW)
                def _():
                    wait_idx(nb)
                    @pl.when(j >= 1)
                    def _():
                        wait_out(nb)
                    start_gather(nb)
                wait_gather(b)
                @pl.when(j + 2 < FPW)
                def _():
                    start_idx(f + 2, b)
                combine(b)
                start_out(f, b)

            half(j, 0)
            half(j + 1, 1)
            return carry

        lax.fori_loop(0, FPW // 2, pair_body, 0)
        wait_out(0)
        wait_out(1)

    k = pl.kernel(
        body,
        out_type=jax.ShapeDtypeStruct((B * T, C), jnp.float32),
        mesh=mesh,
        compiler_params=pltpu.CompilerParams(use_tc_tiling_on_sc=True),
        scratch_types=[
            pltpu.VMEM((CH, 100), jnp.int32),
            pltpu.VMEM((CH, 100), jnp.int32),
            pltpu.VMEM((T, 2 * C), jnp.float32),
            pltpu.VMEM((T, 2 * C), jnp.float32),
            pltpu.VMEM((T, C), jnp.float32),
            pltpu.VMEM((T, C), jnp.float32),
            pltpu.VMEM((T, C), jnp.float32),
            pltpu.SemaphoreType.DMA,
            pltpu.SemaphoreType.DMA,
            pltpu.SemaphoreType.DMA,
            pltpu.SemaphoreType.DMA,
            pltpu.SemaphoreType.DMA,
            pltpu.SemaphoreType.DMA,
        ],
    )
    return k(fused2, idx2, pos)


def kernel(idx, symbol_table, concept_table, law_table, pos_table,
           alpha_logit, beta_logit, W1, b1, W2, b2):
    B, T = idx.shape
    V, C = symbol_table.shape
    fused2 = _make_fused_table(symbol_table, concept_table, law_table,
                               alpha_logit, beta_logit, W1, b1, W2, b2)
    mesh = plsc.VectorSubcoreMesh(core_axis_name="c", subcore_axis_name="s")
    NW = mesh.num_cores * mesh.num_subcores
    assert T % 100 == 0 and B % NW == 0 and C % 16 == 0
    idx2 = idx.reshape(B * T // 100, 100)
    pos = pos_table[:T]
    out = _gather_pos(fused2, idx2, pos, B, T, C, NW, mesh)
    return out.reshape(B, T, C)
